# Initial kernel scaffold; baseline (speedup 1.0000x reference)
import jax
import jax.numpy as jnp
from jax.experimental import pallas as pl


def kernel(x, gate_w, w13, w2, msg_w, q_w, k_w, upd_w1, upd_w2, o_w):
    raise NotImplementedError("write your pallas kernel here")


# speedup vs baseline: 2.3055x; 2.3055x over previous
import math

import jax
import jax.numpy as jnp
from jax.experimental import pallas as pl

_B, _T, _D = 1, 2048, 768
_E, _K = 8, 2
_H = 2048
_CAP = 1024
_AUX_W, _Z_W, _DROP_W = 0.01, 0.001, 0.001
_N = _B * _T
_NK = _N * _K

_TM = 256
_TN = 256


def _ffn_body(x_ref, w13_ref, w2_ref, o_ref):
    xb = x_ref[0]
    z = xb + xb
    gu = jnp.dot(z, w13_ref[0], preferred_element_type=jnp.float32)
    act = jax.nn.silu(gu[:, :_H]) * gu[:, _H:]
    o_ref[0] = xb + jnp.dot(act, w2_ref[0], preferred_element_type=jnp.float32)


def _expert_ffn_pallas(x_cap, w13, w2):
    return pl.pallas_call(
        _ffn_body,
        grid=(_E, _CAP // _TM),
        in_specs=[
            pl.BlockSpec((1, _TM, _D), lambda e, t: (e, t, 0)),
            pl.BlockSpec((1, _D, 2 * _H), lambda e, t: (e, 0, 0)),
            pl.BlockSpec((1, _H, _D), lambda e, t: (e, 0, 0)),
        ],
        out_specs=pl.BlockSpec((1, _TM, _D), lambda e, t: (e, t, 0)),
        out_shape=jax.ShapeDtypeStruct((_E, _CAP, _D), jnp.float32),
    )(x_cap, w13, w2)


def _collab_body(sel_ref, aux_ref, msg_wt_ref, q_wt_ref, k_wt_ref,
                 w1t_ref, w2t_ref, o_wt_ref, out_ref):
    sel2 = sel_ref[...]
    M = jnp.dot(sel2, msg_wt_ref[...], preferred_element_type=jnp.float32)
    Q = jnp.dot(sel2, q_wt_ref[...], preferred_element_type=jnp.float32)
    Kk = jnp.dot(M, k_wt_ref[...], preferred_element_type=jnp.float32)
    QR = Q.reshape(_TN, _K, _D)
    KR = Kk.reshape(_TN, _K, _D)
    MR = M.reshape(_TN, _K, _D)
    kms = (aux_ref[:, 0:1], aux_ref[:, 1:2])
    gts = (aux_ref[:, 2:3], aux_ref[:, 3:4])
    inv = 1.0 / math.sqrt(_D)
    neg = jnp.finfo(jnp.float32).min

    def sc(i, j):
        raw = jnp.sum(QR[:, i, :] * KR[:, j, :], axis=-1, keepdims=True) * inv
        return jnp.where(kms[i] * kms[j] > 0, raw, neg)

    s = [[sc(i, j) for j in range(_K)] for i in range(_K)]
    msgs = []
    for i in range(_K):
        m = jnp.maximum(s[i][0], s[i][1])
        e0 = jnp.exp(s[i][0] - m)
        e1 = jnp.exp(s[i][1] - m)
        dn = e0 + e1
        a0 = e0 / dn * kms[i]
        a1 = e1 / dn * kms[i]
        msgs.append(a0 * MR[:, 0, :] + a1 * MR[:, 1, :])
    msgR = jnp.stack(msgs, axis=1).reshape(_TN * _K, _D)
    upd_in = jnp.concatenate([sel2, msgR], axis=-1)
    pre = jnp.dot(upd_in, w1t_ref[...], preferred_element_type=jnp.float32)
    h1 = 0.5 * pre * (1.0 + jax.lax.erf(pre * (1.0 / math.sqrt(2.0))))
    h = jnp.dot(h1, w2t_ref[...], preferred_element_type=jnp.float32)
    upd = sel2 + h
    UR = upd.reshape(_TN, _K, _D)
    y_tok = gts[0] * UR[:, 0, :] + gts[1] * UR[:, 1, :]
    out_ref[...] = jnp.dot(y_tok, o_wt_ref[...], preferred_element_type=jnp.float32)


def _collab_pallas(sel, aux_tok, msg_wt, q_wt, k_wt, w1t, w2t, o_wt):
    wspec = lambda shape: pl.BlockSpec(shape, lambda t: (0, 0))
    return pl.pallas_call(
        _collab_body,
        grid=(_N // _TN,),
        in_specs=[
            pl.BlockSpec((_K * _TN, _D), lambda t: (t, 0)),
            pl.BlockSpec((_TN, 8), lambda t: (t, 0)),
            wspec((_D, _D)),
            wspec((_D, _D)),
            wspec((_D, _D)),
            wspec((2 * _D, 2 * _D)),
            wspec((2 * _D, _D)),
            wspec((_D, _D)),
        ],
        out_specs=pl.BlockSpec((_TN, _D), lambda t: (t, 0)),
        out_shape=jax.ShapeDtypeStruct((_N, _D), jnp.float32),
    )(sel, aux_tok, msg_wt, q_wt, k_wt, w1t, w2t, o_wt)


def kernel(x, gate_w, w13, w2, msg_w, q_w, k_w, upd_w1, upd_w2, o_w):
    xf = x.reshape(_N, _D)
    logits = xf @ gate_w.T
    topk_vals, topk_idx = jax.lax.top_k(logits, _K)
    topk_probs = jax.nn.softmax(topk_vals, axis=-1)
    router_probs = jax.nn.softmax(logits, axis=-1)
    assign = jnp.zeros((_N, _E), jnp.float32).at[
        jnp.arange(_N)[:, None], topk_idx].add(topk_probs)
    balance = (router_probs.mean(0) * assign.mean(0)).sum() * _E
    zlse = jax.nn.logsumexp(logits, axis=-1)
    aux = _AUX_W * balance + _Z_W * (zlse * zlse).mean()

    target = topk_idx.reshape(-1)
    prio = topk_vals.reshape(-1)
    order = jnp.lexsort((-prio, target))
    counts = jnp.bincount(target, length=_E)
    starts = jnp.concatenate(
        [jnp.zeros((1,), counts.dtype), jnp.cumsum(counts)[:-1]])

    c_grid = jnp.arange(_CAP)[None, :]
    pos = starts[:, None] + c_grid
    valid = c_grid < jnp.minimum(counts, _CAP)[:, None]
    r = order[jnp.clip(pos, 0, _NK - 1)]
    tok = r // _K
    x_cap = jnp.where(valid[..., None], xf[tok], 0.0)

    y_cap = _expert_ffn_pallas(x_cap, w13, w2)

    inv_order = jnp.zeros((_NK,), jnp.int32).at[order].set(
        jnp.arange(_NK, dtype=jnp.int32))
    wr = inv_order - starts[target]
    keptr = wr < _CAP
    slot = target * _CAP + jnp.clip(wr, 0, _CAP - 1)
    y_assign = jnp.where(keptr[:, None],
                         y_cap.reshape(_E * _CAP, _D)[slot], 0.0)

    drop_frac = 1.0 - keptr.astype(jnp.float32).mean()
    aux = aux + _DROP_W * drop_frac

    km = keptr.reshape(_N, _K).astype(jnp.float32)
    gts = topk_probs * km
    aux_tok = jnp.concatenate(
        [km, gts, jnp.zeros((_N, 4), jnp.float32)], axis=1)

    y = _collab_pallas(y_assign, aux_tok, msg_w.T, q_w.T, k_w.T,
                       upd_w1.T, upd_w2.T, o_w.T)
    return y.reshape(_B, _T, _D), aux, topk_idx.reshape(_B, _T, _K)
